# baseline (speedup 1.0000x reference)
import functools
import math

import jax
import jax.numpy as jnp
from jax import lax
from jax.experimental import pallas as pl
from jax.experimental.pallas import tpu as pltpu
from jax.experimental.pallas import tpu_sc as plsc

D_LANES = 16

_info = plsc.get_sparse_core_info()
NUM_CORES = _info.num_cores
NUM_SUBCORES = _info.num_subcores
NUM_WORKERS = NUM_CORES * NUM_SUBCORES

CHUNK = 512


def _make_lookup(B, V, D):
    assert B % NUM_WORKERS == 0
    b_per_w = B // NUM_WORKERS
    assert b_per_w % CHUNK == 0
    n_chunks = b_per_w // CHUNK
    scale = math.sqrt(D)
    n_vecs = D // D_LANES
    mesh = plsc.VectorSubcoreMesh(core_axis_name="c", subcore_axis_name="s")

    @functools.partial(
        pl.kernel,
        mesh=mesh,
        compiler_params=pltpu.CompilerParams(use_tc_tiling_on_sc=False),
        out_type=jax.ShapeDtypeStruct((B, D), jnp.float32),
        scratch_types=[
            pltpu.VMEM((CHUNK,), jnp.int32),
            pltpu.VMEM((CHUNK, D), jnp.float32),
            pltpu.SemaphoreType.DMA,
        ],
    )
    def lookup(idx_hbm, lut_hbm, out_hbm, idx_v, rows_v, sem):
        wid = lax.axis_index("s") * NUM_CORES + lax.axis_index("c")
        base = wid * b_per_w

        def chunk_body(c, carry):
            off = base + c * CHUNK
            pltpu.sync_copy(idx_hbm.at[pl.ds(off, CHUNK)], idx_v)
            pltpu.async_copy(lut_hbm.at[idx_v], rows_v, sem).wait()

            def row_body(i, carry2):
                for j in range(n_vecs):
                    sl = pl.ds(j * D_LANES, D_LANES)
                    rows_v[i, sl] = rows_v[i, sl] * scale
                return carry2

            lax.fori_loop(0, CHUNK, row_body, 0)
            pltpu.sync_copy(rows_v, out_hbm.at[pl.ds(off, CHUNK)])
            return carry

        lax.fori_loop(0, n_chunks, chunk_body, 0)

    return lookup


def kernel(x, lut):
    B = x.size
    V, D = lut.shape
    flat_idx = x.reshape(-1).astype(jnp.int32)
    out = _make_lookup(B, V, D)(flat_idx, lut)
    return out.reshape(x.shape + (D,))


# speedup vs baseline: 1.1330x; 1.1330x over previous
import functools
import math

import jax
import jax.numpy as jnp
from jax import lax
from jax.experimental import pallas as pl
from jax.experimental.pallas import tpu as pltpu
from jax.experimental.pallas import tpu_sc as plsc

LANES = 16

_info = plsc.get_sparse_core_info()
NUM_CORES = _info.num_cores
NUM_SUBCORES = _info.num_subcores
NUM_WORKERS = NUM_CORES * NUM_SUBCORES

CHUNK = 256
NBUF = 4
PREF = 2
ROWS_PER_STEP = 4


def _make_lookup(B, V, D):
    assert B % NUM_WORKERS == 0
    b_per_w = B // NUM_WORKERS
    assert b_per_w % (CHUNK * NBUF) == 0
    n_chunks = b_per_w // CHUNK
    n_groups = n_chunks // NBUF
    scale = math.sqrt(D)
    n_vecs = D // LANES
    mesh = plsc.VectorSubcoreMesh(core_axis_name="c", subcore_axis_name="s")

    @functools.partial(
        pl.kernel,
        mesh=mesh,
        compiler_params=pltpu.CompilerParams(use_tc_tiling_on_sc=False),
        out_type=jax.ShapeDtypeStruct((B, D), jnp.float32),
        scratch_types=[
            pltpu.VMEM((b_per_w,), jnp.int32),
            [pltpu.VMEM((CHUNK, D), jnp.float32) for _ in range(NBUF)],
            [pltpu.SemaphoreType.DMA for _ in range(NBUF)],
            [pltpu.SemaphoreType.DMA for _ in range(NBUF)],
        ],
    )
    def lookup(idx_hbm, lut_hbm, out_hbm, idx_v, rows, gsem, ssem):
        wid = lax.axis_index("s") * NUM_CORES + lax.axis_index("c")
        base = wid * b_per_w
        pltpu.sync_copy(idx_hbm.at[pl.ds(base, b_per_w)], idx_v)

        def gather_copy(c, b):
            return pltpu.make_async_copy(
                lut_hbm.at[idx_v.at[pl.ds(c * CHUNK, CHUNK)]], rows[b], gsem[b]
            )

        def scatter_copy(c, b):
            return pltpu.make_async_copy(
                rows[b], out_hbm.at[pl.ds(base + c * CHUNK, CHUNK)], ssem[b]
            )

        for c0 in range(PREF):
            gather_copy(c0, c0).start()

        def group_body(g, carry):
            for b in range(NBUF):
                c = g * NBUF + b
                pb = (b + PREF) % NBUF
                p = c + PREF
                if b + PREF < NBUF:
                    @pl.when(g > 0)
                    def _():
                        scatter_copy(p - NBUF, pb).wait()

                    gather_copy(p, pb).start()
                else:
                    @pl.when(g < n_groups - 1)
                    def _():
                        scatter_copy(p - NBUF, pb).wait()
                        gather_copy(p, pb).start()

                gather_copy(c, b).wait()

                r = rows[b]

                @plsc.parallel_loop(0, CHUNK, step=ROWS_PER_STEP, unroll=2)
                def _(i):
                    for ii in range(ROWS_PER_STEP):
                        for j in range(n_vecs):
                            sl = pl.ds(j * LANES, LANES)
                            r[i + ii, sl] = r[i + ii, sl] * scale

                scatter_copy(c, b).start()
            return carry

        lax.fori_loop(0, n_groups, group_body, 0)

        for b in range(NBUF):
            scatter_copy(n_chunks - NBUF + b, b).wait()

    return lookup


def kernel(x, lut):
    B = x.size
    V, D = lut.shape
    flat_idx = x.reshape(-1).astype(jnp.int32)
    out = _make_lookup(B, V, D)(flat_idx, lut)
    return out.reshape(x.shape + (D,))
